# Initial kernel scaffold; baseline (speedup 1.0000x reference)
import jax
import jax.numpy as jnp
from jax.experimental import pallas as pl


def kernel(mem, mem_ts, node_ids, messages, timestamps):
    raise NotImplementedError("write your pallas kernel here")


# speedup vs baseline: 7.3585x; 7.3585x over previous
import functools

import jax
import jax.numpy as jnp
from jax import lax
from jax.experimental import pallas as pl
from jax.experimental.pallas import tpu as pltpu
from jax.experimental.pallas import tpu_sc as plsc

NC, NS, L = 2, 16, 16
NW = NC * NS


def _sc_body(M, B, D, nid_hbm, msg_hbm, ts_hbm, newmem_hbm, newts_hbm,
             has_hbm, nid_v, table_v, t2d_v, dst2d_v, t128_v, dst128_v,
             tsval_v, ones_v, rowbuf0_v, sem0, sem1):
    wid = lax.axis_index("s") * NC + lax.axis_index("c")
    chunk = B // NW
    base = wid * chunk
    iota = lax.iota(jnp.int32, L)
    full = iota >= 0
    nvec = B // L

    pltpu.sync_copy(nid_hbm, nid_v)

    def scat(j, c):
        ids = nid_v[pl.ds(j * L, L)]
        pos = j * L + iota
        plsc.store_scatter(table_v, [ids], pos, mask=full)
        return c

    lax.fori_loop(0, nvec, scat, 0)

    def sweep(_):
        def fix(j, acc):
            ids = nid_v[pl.ds(j * L, L)]
            pos = j * L + iota
            t = plsc.load_gather(table_v, [ids], mask=full)
            m = pos > t
            plsc.store_scatter(table_v, [ids], pos, mask=m)
            return acc + jnp.where(m, 1, 0)

        acc = lax.fori_loop(0, nvec, fix, jnp.zeros((L,), jnp.int32))
        return jnp.max(acc)

    c0 = sweep(0)
    lax.while_loop(lambda c: c > 0, sweep, c0)

    for k in range(8):
        ones_v[pl.ds(k * L, L)] = jnp.ones((L,), jnp.int32)

    n16 = chunk // L
    for j in range(n16):
        ids = nid_v[pl.ds(base + j * L, L)]
        t = plsc.load_gather(table_v, [ids], mask=full)
        t2d_v[j] = t
        dst2d_v[j] = ids
        t128_v[j // 8, pl.ds((j % 8) * L, L)] = t
        dst128_v[j // 8, pl.ds((j % 8) * L, L)] = ids

    for q in range(chunk // 128):
        pltpu.async_copy(ts_hbm.at[t128_v.at[q]], tsval_v.at[q], sem0).wait()
        pltpu.sync_copy(tsval_v.at[q], newts_hbm.at[dst128_v.at[q]])
        pltpu.sync_copy(ones_v, has_hbm.at[dst128_v.at[q]])

    for k in range(n16):
        pltpu.async_copy(msg_hbm.at[t2d_v.at[k]], rowbuf0_v, sem1).wait()
        pltpu.sync_copy(rowbuf0_v, newmem_hbm.at[dst2d_v.at[k]])


def _make_sc_call(M, B, D, interpret=False):
    chunk = B // NW
    mesh = plsc.VectorSubcoreMesh(core_axis_name="c", subcore_axis_name="s",
                                  num_cores=NC, num_subcores=NS)
    return pl.kernel(
        functools.partial(_sc_body, M, B, D),
        out_type=(),
        mesh=mesh,
        scratch_types=[
            pltpu.VMEM((B,), jnp.int32),
            pltpu.VMEM((M,), jnp.int32),
            pltpu.VMEM((chunk // L, L), jnp.int32),
            pltpu.VMEM((chunk // L, L), jnp.int32),
            pltpu.VMEM((chunk // 128, 128), jnp.int32),
            pltpu.VMEM((chunk // 128, 128), jnp.int32),
            pltpu.VMEM((chunk // 128, 128), jnp.float32),
            pltpu.VMEM((128,), jnp.int32),
            pltpu.VMEM((L, D), jnp.float32),
            pltpu.SemaphoreType.DMA,
            pltpu.SemaphoreType.DMA,
        ],
        interpret=interpret,
        compiler_params=pltpu.CompilerParams(needs_layout_passes=False),
        name="message_aggregator_sc",
    )


def kernel(mem, mem_ts, node_ids, messages, timestamps):
    M, D = mem.shape
    B = node_ids.shape[0]
    newmem = jax.new_ref(mem)
    newts = jax.new_ref(mem_ts)
    has = jax.new_ref(jnp.zeros((M,), jnp.int32))
    _make_sc_call(M, B, D)(node_ids, messages, timestamps, newmem, newts, has)
    return newmem[...], newts[...], has[...].astype(jnp.bool_)


# speedup vs baseline: 8.5385x; 1.1604x over previous
import functools

import jax
import jax.numpy as jnp
from jax import lax
from jax.experimental import pallas as pl
from jax.experimental.pallas import tpu as pltpu
from jax.experimental.pallas import tpu_sc as plsc

NC, NS, L = 2, 16, 16
NW = NC * NS
CH = 2048
RPS = 16


def _sc_body(M, B, D, nid_hbm, msg_hbm, ts_hbm, newmem_hbm, newts_hbm,
             has_hbm, nidc0_v, nidc1_v, nid512_v, table_v, t2d_v, dst2d_v,
             t128_v, dst128_v, tsval_v, ones_v, rowbuf0_v, rowbuf1_v,
             semg0, semg1, sems0, sems1):
    wid = lax.axis_index("s") * NC + lax.axis_index("c")
    chunk = B // NW
    base = wid * chunk
    iota = lax.iota(jnp.int32, L)
    full = iota >= 0

    pltpu.sync_copy(nid_hbm.at[pl.ds(base, chunk)], nid512_v)
    for k in range(8):
        ones_v[pl.ds(k * L, L)] = jnp.ones((L,), jnp.int32)
    for j in range(chunk // L):
        ids = nid512_v[pl.ds(j * L, L)]
        dst2d_v[j] = ids
        dst128_v[j // 8, pl.ds((j % 8) * L, L)] = ids
    for q in range(chunk // 128):
        pltpu.sync_copy(ones_v, has_hbm.at[dst128_v.at[q]])

    nblk = B // CH
    bufs = (nidc0_v, nidc1_v)
    sems = (semg0, semg1)
    cps = [None, None]
    cps[0] = pltpu.async_copy(nid_hbm.at[pl.ds(0, CH)], bufs[0], sems[0])

    UNROLL = 8

    def scat_group(buf, blk):
        def body(j, c):
            idss = [buf[pl.ds((j * UNROLL + u) * L, L)] for u in range(UNROLL)]
            wins = [plsc.scan_count(ids)[1] for ids in idss]
            for u in range(UNROLL):
                pos = blk * CH + (j * UNROLL + u) * L + iota
                plsc.store_scatter(table_v, [idss[u]], pos, mask=wins[u])
            return c

        lax.fori_loop(0, CH // (L * UNROLL), body, 0)

    for blk in range(nblk):
        cps[blk % 2].wait()
        if blk + 1 < nblk:
            cps[(blk + 1) % 2] = pltpu.async_copy(
                nid_hbm.at[pl.ds((blk + 1) * CH, CH)], bufs[(blk + 1) % 2],
                sems[(blk + 1) % 2])
        scat_group(bufs[blk % 2], blk)

    for j in range(chunk // L):
        ids = nid512_v[pl.ds(j * L, L)]
        t = plsc.load_gather(table_v, [ids], mask=full)
        t2d_v[j] = t
        t128_v[j // 8, pl.ds((j % 8) * L, L)] = t

    for q in range(chunk // 128):
        pltpu.async_copy(ts_hbm.at[t128_v.at[q]], tsval_v.at[q], semg0).wait()
        pltpu.sync_copy(tsval_v.at[q], newts_hbm.at[dst128_v.at[q]])

    n16 = chunk // RPS
    rbufs = (rowbuf0_v, rowbuf1_v)
    gsems = (semg0, semg1)
    ssems = (sems0, sems1)
    g = [None, None]
    s = [None, None]
    g[0] = pltpu.async_copy(msg_hbm.at[t2d_v.at[0]], rbufs[0], gsems[0])
    for k in range(n16):
        b = k % 2
        b2 = (k + 1) % 2
        if k + 1 < n16:
            if s[b2] is not None:
                s[b2].wait()
            g[b2] = pltpu.async_copy(msg_hbm.at[t2d_v.at[k + 1]], rbufs[b2],
                                     gsems[b2])
        g[b].wait()
        s[b] = pltpu.async_copy(rbufs[b], newmem_hbm.at[dst2d_v.at[k]],
                                ssems[b])
    s[0].wait()
    s[1].wait()


def _make_sc_call(M, B, D, interpret=False):
    chunk = B // NW
    mesh = plsc.VectorSubcoreMesh(core_axis_name="c", subcore_axis_name="s",
                                  num_cores=NC, num_subcores=NS)
    return pl.kernel(
        functools.partial(_sc_body, M, B, D),
        out_type=(),
        mesh=mesh,
        scratch_types=[
            pltpu.VMEM((CH,), jnp.int32),
            pltpu.VMEM((CH,), jnp.int32),
            pltpu.VMEM((chunk,), jnp.int32),
            pltpu.VMEM((M,), jnp.int32),
            pltpu.VMEM((chunk // L, L), jnp.int32),
            pltpu.VMEM((chunk // L, L), jnp.int32),
            pltpu.VMEM((chunk // 128, 128), jnp.int32),
            pltpu.VMEM((chunk // 128, 128), jnp.int32),
            pltpu.VMEM((chunk // 128, 128), jnp.float32),
            pltpu.VMEM((128,), jnp.int32),
            pltpu.VMEM((RPS, D), jnp.float32),
            pltpu.VMEM((RPS, D), jnp.float32),
            pltpu.SemaphoreType.DMA,
            pltpu.SemaphoreType.DMA,
            pltpu.SemaphoreType.DMA,
            pltpu.SemaphoreType.DMA,
        ],
        interpret=interpret,
        compiler_params=pltpu.CompilerParams(needs_layout_passes=False),
        name="message_aggregator_sc",
    )


def kernel(mem, mem_ts, node_ids, messages, timestamps):
    M, D = mem.shape
    B = node_ids.shape[0]
    newmem = jax.new_ref(mem)
    newts = jax.new_ref(mem_ts)
    has = jax.new_ref(jnp.zeros((M,), jnp.int32))
    _make_sc_call(M, B, D)(node_ids, messages, timestamps, newmem, newts, has)
    return newmem[...], newts[...], has[...].astype(jnp.bool_)
